# Initial kernel scaffold; baseline (speedup 1.0000x reference)
import jax
import jax.numpy as jnp
from jax.experimental import pallas as pl


def kernel(logits, top_k):
    raise NotImplementedError("write your pallas kernel here")


# speedup vs baseline: 2.0490x; 2.0490x over previous
import functools

import jax
import jax.numpy as jnp
from jax import lax
from jax.experimental import pallas as pl
from jax.experimental.pallas import tpu as pltpu

_TEMPERATURE = 0.8
_KMAX = 50


def _f32_to_key(x):
    i = lax.bitcast_convert_type(x, jnp.int32)
    return i ^ ((i >> 31) & jnp.int32(0x7FFFFFFF))


def _key_to_f32(k):
    i = k ^ ((k >> 31) & jnp.int32(0x7FFFFFFF))
    return lax.bitcast_convert_type(i, jnp.float32)


def _body(tk_ref, x_ref, g_ref, probs_ref, idx_ref):
    y = x_ref[...] / jnp.float32(_TEMPERATURE)
    br, v = y.shape
    k_sel = jnp.clip(tk_ref[0], 1, _KMAX)

    rmax = jnp.max(y, axis=1, keepdims=True)
    rmin = jnp.min(y, axis=1, keepdims=True)

    lo0 = _f32_to_key(rmin)
    hi0 = _f32_to_key(rmax)

    def cond(c):
        lo, hi = c
        return jnp.any(lo < hi)

    def step(c):
        lo, hi = c
        d = lax.bitcast_convert_type(hi - lo, jnp.uint32)
        half = lax.bitcast_convert_type((d + jnp.uint32(1)) >> 1, jnp.int32)
        mid = lo + half
        t = _key_to_f32(mid)
        cnt = jnp.sum((y >= t).astype(jnp.int32), axis=1, keepdims=True)
        pred = cnt >= k_sel
        return jnp.where(pred, mid, lo), jnp.where(pred, hi, mid - 1)

    lo, _ = lax.while_loop(cond, step, (lo0, hi0))
    thresh = _key_to_f32(lo)

    keep = y >= thresh
    s = jnp.where(keep, jnp.exp(y - rmax), jnp.float32(0.0))
    denom = jnp.sum(s, axis=1, keepdims=True)
    probs_ref[...] = s / denom

    z = jnp.where(keep, y + g_ref[...], -jnp.inf)
    zmax = jnp.max(z, axis=1, keepdims=True)
    cols = lax.broadcasted_iota(jnp.int32, (br, v), 1)
    idx = jnp.min(jnp.where(z == zmax, cols, jnp.int32(0x7FFFFFFF)), axis=1)
    idx_ref[...] = idx.reshape(br, 1)


@jax.jit
def kernel(logits, top_k):
    r, v = logits.shape
    br = 8
    g = jax.random.gumbel(jax.random.key(1), (r, v), jnp.float32)
    tk = jnp.asarray(top_k, jnp.int32).reshape(1)

    grid_spec = pltpu.PrefetchScalarGridSpec(
        num_scalar_prefetch=1,
        grid=(r // br,),
        in_specs=[
            pl.BlockSpec((br, v), lambda i, tk_ref: (i, 0)),
            pl.BlockSpec((br, v), lambda i, tk_ref: (i, 0)),
        ],
        out_specs=[
            pl.BlockSpec((br, v), lambda i, tk_ref: (i, 0)),
            pl.BlockSpec((br, 1), lambda i, tk_ref: (i, 0)),
        ],
    )
    probs, idx = pl.pallas_call(
        _body,
        grid_spec=grid_spec,
        out_shape=[
            jax.ShapeDtypeStruct((r, v), jnp.float32),
            jax.ShapeDtypeStruct((r, 1), jnp.int32),
        ],
    )(tk, logits, g)
    return probs, idx[:, 0]


# speedup vs baseline: 3.6292x; 1.7712x over previous
import functools

import jax
import jax.numpy as jnp
from jax import lax
from jax.experimental import pallas as pl
from jax.experimental.pallas import tpu as pltpu

_TEMPERATURE = 0.8
_KMAX = 50


def _f32_to_key(x):
    i = lax.bitcast_convert_type(x, jnp.int32)
    return i ^ ((i >> 31) & jnp.int32(0x7FFFFFFF))


def _key_to_f32(k):
    i = k ^ ((k >> 31) & jnp.int32(0x7FFFFFFF))
    return lax.bitcast_convert_type(i, jnp.float32)


def _kth_largest_search(data, k_sel, lo0, hi0):
    def cond(c):
        lo, hi = c
        return jnp.any(lo < hi)

    def step(c):
        lo, hi = c
        d = lax.bitcast_convert_type(hi - lo, jnp.uint32)
        half = lax.bitcast_convert_type((d + jnp.uint32(1)) >> 1, jnp.int32)
        mid = lo + half
        t = _key_to_f32(mid)
        cnt = jnp.sum((data >= t).astype(jnp.int32), axis=1, keepdims=True)
        pred = cnt >= k_sel
        return jnp.where(pred, mid, lo), jnp.where(pred, hi, mid - 1)

    lo, _ = lax.while_loop(cond, step, (lo0, hi0))
    return _key_to_f32(lo)


def _per_lane_top4(ypad):
    mx, mn = jnp.maximum, jnp.minimum
    a, b = ypad[:, :512], ypad[:, 512:]
    s1, s2 = mx(a, b), mn(a, b)
    a1, a2 = s1[:, :256], s2[:, :256]
    b1, b2 = s1[:, 256:], s2[:, 256:]
    o1, o4 = mx(a1, b1), mn(a2, b2)
    t1, t2 = mn(a1, b1), mx(a2, b2)
    lists = (o1, mx(t1, t2), mn(t1, t2), o4)
    n = 256
    while n > 1:
        h = n // 2
        a1, a2, a3, a4 = (l[:, :h] for l in lists)
        b1, b2, b3, b4 = (l[:, h:] for l in lists)
        h1, h2, h3, h4 = mx(a1, b4), mx(a2, b3), mx(a3, b2), mx(a4, b1)
        p1, p3 = mx(h1, h3), mn(h1, h3)
        p2, p4 = mx(h2, h4), mn(h2, h4)
        lists = (mx(p1, p2), mn(p1, p2), mx(p3, p4), mn(p3, p4))
        n = h
    return tuple(l[:, 0] for l in lists)


def _body(tk_ref, x_ref, g_ref, probs_ref, idx_ref):
    y = x_ref[...] / jnp.float32(_TEMPERATURE)
    br, v = y.shape
    k_sel = jnp.clip(tk_ref[0], 1, _KMAX)
    neg_inf = jnp.float32(-jnp.inf)

    rmax = jnp.max(y, axis=1, keepdims=True)

    vfull = (v // 128) * 128
    ya = y[:, :vfull].reshape(br, vfull // 128, 128)
    rem = v - vfull
    if rem:
        yb = jnp.concatenate(
            [y[:, vfull:], jnp.full((br, 128 - rem), neg_inf, jnp.float32)],
            axis=1,
        ).reshape(br, 1, 128)
        ya = jnp.concatenate([ya, yb], axis=1)
    npad = 1024 - ya.shape[1]
    ypad = jnp.concatenate(
        [ya, jnp.full((br, npad, 128), neg_inf, jnp.float32)], axis=1
    )
    m1, m2, m3, m4 = _per_lane_top4(ypad)
    cand = jnp.concatenate([m1, m2, m3, m4], axis=1)

    lo0 = _f32_to_key(jnp.min(m4, axis=1, keepdims=True))
    hi0 = _f32_to_key(rmax)
    t_c = _kth_largest_search(cand, k_sel, lo0, hi0)

    cnt_gt = jnp.sum((y > t_c).astype(jnp.int32), axis=1, keepdims=True)
    ok = cnt_gt < k_sel

    thresh = lax.cond(
        jnp.all(ok),
        lambda: t_c,
        lambda: _kth_largest_search(y, k_sel, lo0, hi0),
    )

    keep = y >= thresh
    s = jnp.where(keep, jnp.exp(y - rmax), jnp.float32(0.0))
    denom = jnp.sum(s, axis=1, keepdims=True)
    probs_ref[...] = s * (jnp.float32(1.0) / denom)

    z = jnp.where(keep, y + g_ref[...], neg_inf)
    zmax = jnp.max(z, axis=1, keepdims=True)
    cols = lax.broadcasted_iota(jnp.int32, (br, v), 1)
    idx = jnp.min(jnp.where(z == zmax, cols, jnp.int32(0x7FFFFFFF)), axis=1)
    idx_ref[...] = idx.reshape(br, 1)


@jax.jit
def kernel(logits, top_k):
    r, v = logits.shape
    br = 8
    g = jax.random.gumbel(jax.random.key(1), (r, v), jnp.float32)
    tk = jnp.asarray(top_k, jnp.int32).reshape(1)

    grid_spec = pltpu.PrefetchScalarGridSpec(
        num_scalar_prefetch=1,
        grid=(r // br,),
        in_specs=[
            pl.BlockSpec((br, v), lambda i, tk_ref: (i, 0)),
            pl.BlockSpec((br, v), lambda i, tk_ref: (i, 0)),
        ],
        out_specs=[
            pl.BlockSpec((br, v), lambda i, tk_ref: (i, 0)),
            pl.BlockSpec((br, 1), lambda i, tk_ref: (i, 0)),
        ],
    )
    probs, idx = pl.pallas_call(
        _body,
        grid_spec=grid_spec,
        out_shape=[
            jax.ShapeDtypeStruct((r, v), jnp.float32),
            jax.ShapeDtypeStruct((r, 1), jnp.int32),
        ],
    )(tk, logits, g)
    return probs, idx[:, 0]
